# baseline (speedup 1.0000x reference)
import functools

import jax
import jax.numpy as jnp
from jax import lax
from jax.experimental import pallas as pl
from jax.experimental.pallas import tpu as pltpu
from jax.experimental.pallas import tpu_sc as plsc

INPUT_DIM = 2048
BOTTLENECK = 16384
K = 32


def _enc_body_f32(w_ref, x_ref, b_ref, z1_ref):
    z = jax.lax.dot_general(
        x_ref[...], w_ref[...],
        dimension_numbers=(((1,), (1,)), ((), ())),
        preferred_element_type=jnp.float32)
    z1_ref[...] = z + b_ref[...]


def _encoder_f32(x, W, b2d, tok_blk, bn_blk):
    n_tok = x.shape[0]
    grid = (BOTTLENECK // bn_blk, n_tok // tok_blk)
    return pl.pallas_call(
        _enc_body_f32,
        grid=grid,
        in_specs=[
            pl.BlockSpec((bn_blk, INPUT_DIM), lambda j, i: (j, 0)),
            pl.BlockSpec((tok_blk, INPUT_DIM), lambda j, i: (i, 0)),
            pl.BlockSpec((1, bn_blk), lambda j, i: (0, j)),
        ],
        out_specs=pl.BlockSpec((tok_blk, bn_blk), lambda j, i: (i, j)),
        out_shape=jax.ShapeDtypeStruct((n_tok, BOTTLENECK), jnp.float32),
        compiler_params=pltpu.CompilerParams(
            dimension_semantics=("arbitrary", "arbitrary"),
        ),
    )(W, x, b2d)


def _enc_body(wh_ref, wl_ref, xh_ref, xl_ref, b_ref, z1_ref):
    dn = (((1,), (1,)), ((), ()))
    z = jax.lax.dot_general(
        xh_ref[...], wh_ref[...], dimension_numbers=dn,
        preferred_element_type=jnp.float32)
    z += jax.lax.dot_general(
        xh_ref[...], wl_ref[...], dimension_numbers=dn,
        preferred_element_type=jnp.float32)
    z += jax.lax.dot_general(
        xl_ref[...], wh_ref[...], dimension_numbers=dn,
        preferred_element_type=jnp.float32)
    z1_ref[...] = z + b_ref[...]


def _encoder(xh, xl, Wh, Wl, b2d, tok_blk, bn_blk):
    n_tok = xh.shape[0]
    grid = (BOTTLENECK // bn_blk, n_tok // tok_blk)
    w_spec = pl.BlockSpec((bn_blk, INPUT_DIM), lambda j, i: (j, 0))
    x_spec = pl.BlockSpec((tok_blk, INPUT_DIM), lambda j, i: (i, 0))
    return pl.pallas_call(
        _enc_body,
        grid=grid,
        in_specs=[w_spec, w_spec, x_spec, x_spec,
                  pl.BlockSpec((1, bn_blk), lambda j, i: (0, j))],
        out_specs=pl.BlockSpec((tok_blk, bn_blk), lambda j, i: (i, j)),
        out_shape=jax.ShapeDtypeStruct((n_tok, BOTTLENECK), jnp.float32),
        compiler_params=pltpu.CompilerParams(
            dimension_semantics=("arbitrary", "arbitrary"),
        ),
    )(Wh, Wl, xh, xl, b2d)


def _topk_body_ablate(z1_ref, a1_ref, fs_ref):
    a1_ref[...] = z1_ref[...].astype(jnp.bfloat16)


def _topk_body(z1_ref, a1_ref, fs_ref, *, out_dtype=jnp.float32,
               exact=True):
    R = z1_ref.shape[0]
    N = z1_ref.shape[1]
    NEG = jnp.float32(-jnp.inf)

    fs_ref[:, : N // 2] = jnp.maximum(z1_ref[:, : N // 2], z1_ref[:, N // 2:])
    w = N // 4
    while w >= 128:
        fs_ref[:, :w] = jnp.maximum(fs_ref[:, :w], fs_ref[:, w:2 * w])
        w //= 2

    def extract(n_iter, width):
        def body(_, t):
            blk = fs_ref[:, :width]
            m = jnp.max(blk, axis=1, keepdims=True)
            fs_ref[:, :width] = jnp.where(blk >= m, NEG, blk)
            return m
        return jax.lax.fori_loop(
            0, n_iter, body, jnp.zeros((R, 1), jnp.float32))

    t_lo = extract(K + 1, 128)

    zlo = z1_ref[:, : N // 2]
    zhi = z1_ref[:, N // 2:]
    fs_ref[:, : N // 2] = jnp.maximum(
        jnp.where(zlo >= t_lo, zlo, NEG), jnp.where(zhi >= t_lo, zhi, NEG))
    w = N // 4
    while w >= 1024:
        fs_ref[:, :w] = jnp.maximum(fs_ref[:, :w], fs_ref[:, w:2 * w])
        w //= 2

    thr = jnp.maximum(extract(K, 1024), t_lo)
    if exact:
        cnt1 = jnp.sum((z1_ref[...] >= thr).astype(jnp.float32),
                       axis=1, keepdims=True)
        kf = jnp.float32(K)

        def cond(carry):
            _, cnt = carry
            return jnp.any(cnt > kf)

        def body(carry):
            t, cnt = carry
            zz = z1_ref[...]
            active = cnt > kf
            m = jnp.min(jnp.where(zz >= t, zz, jnp.inf),
                        axis=1, keepdims=True)
            mb = jax.lax.bitcast_convert_type(m, jnp.uint32)
            up = jnp.where(m >= 0, mb + 1, mb - 1)
            t_next = jax.lax.bitcast_convert_type(up, jnp.float32)
            t_new = jnp.where(active, t_next, t)
            cnt_new = jnp.sum((zz >= t_new).astype(jnp.float32),
                              axis=1, keepdims=True)
            return t_new, cnt_new

        thr, _ = jax.lax.while_loop(cond, body, (thr, cnt1))
    z = z1_ref[...]
    a1_ref[...] = jnp.where(z >= thr, z, 0.0).astype(out_dtype)


def _topk_mask(z1, tok_blk, out_dtype=jnp.float32, exact=True):
    n_tok = z1.shape[0]
    return pl.pallas_call(
        functools.partial(_topk_body, out_dtype=out_dtype, exact=exact),
        grid=(n_tok // tok_blk,),
        in_specs=[pl.BlockSpec((tok_blk, BOTTLENECK), lambda i: (i, 0))],
        out_specs=pl.BlockSpec((tok_blk, BOTTLENECK), lambda i: (i, 0)),
        out_shape=jax.ShapeDtypeStruct((n_tok, BOTTLENECK), out_dtype),
        scratch_shapes=[pltpu.VMEM((tok_blk, BOTTLENECK // 2), jnp.float32)],
        compiler_params=pltpu.CompilerParams(
            dimension_semantics=("arbitrary",),
        ),
    )(z1)


def _dec_body(a1_ref, w_ref, db_ref, z2_ref, acc_ref, *, n_kc):
    kc = pl.program_id(1)

    @pl.when(kc == 0)
    def _():
        acc_ref[...] = jnp.zeros_like(acc_ref)

    acc_ref[...] += jax.lax.dot_general(
        a1_ref[...], w_ref[...],
        dimension_numbers=(((1,), (0,)), ((), ())),
        preferred_element_type=jnp.float32,
    )

    @pl.when(kc == n_kc - 1)
    def _():
        z2_ref[...] = acc_ref[...] + db_ref[...]


def _decoder(a1, W, db2d, tok_blk, kc_blk):
    n_tok = a1.shape[0]
    n_kc = BOTTLENECK // kc_blk
    grid = (n_tok // tok_blk, n_kc)
    return pl.pallas_call(
        functools.partial(_dec_body, n_kc=n_kc),
        grid=grid,
        in_specs=[
            pl.BlockSpec((tok_blk, kc_blk), lambda i, k: (i, k)),
            pl.BlockSpec((kc_blk, INPUT_DIM), lambda i, k: (k, 0)),
            pl.BlockSpec((1, INPUT_DIM), lambda i, k: (0, 0)),
        ],
        out_specs=pl.BlockSpec((tok_blk, INPUT_DIM), lambda i, k: (i, 0)),
        out_shape=jax.ShapeDtypeStruct((n_tok, INPUT_DIM), jnp.float32),
        scratch_shapes=[pltpu.VMEM((tok_blk, INPUT_DIM), jnp.float32)],
        compiler_params=pltpu.CompilerParams(
            dimension_semantics=("arbitrary", "arbitrary"),
        ),
    )(a1, W, db2d)


_NW = 32


def _vbroadcast_lane(v, lane):
    idx = jnp.full((16,), lane, dtype=jnp.int32)
    return lax.gather(
        v, idx[:, None],
        dimension_numbers=lax.GatherDimensionNumbers(
            offset_dims=(), collapsed_slice_dims=(0,), start_index_map=(0,)),
        slice_sizes=(1,),
        mode=lax.GatherScatterMode.PROMISE_IN_BOUNDS)


_CAP = 48


def _sc_decoder(a1, W, db1d):
    n_tok = a1.shape[0]
    tpw = n_tok // _NW
    mesh = plsc.VectorSubcoreMesh(core_axis_name="c", subcore_axis_name="s")

    @functools.partial(
        pl.kernel,
        mesh=mesh,
        out_type=jax.ShapeDtypeStruct((n_tok, INPUT_DIM), jnp.float32),
        scratch_types=[
            pltpu.VMEM((BOTTLENECK,), jnp.float32),
            pltpu.VMEM((_CAP,), jnp.float32),
            pltpu.VMEM((_CAP,), jnp.int32),
            pltpu.VMEM((K,), jnp.float32),
            pltpu.VMEM((K,), jnp.int32),
            pltpu.VMEM((K, INPUT_DIM), jnp.float32),
            pltpu.VMEM((INPUT_DIM,), jnp.float32),
            pltpu.VMEM((INPUT_DIM,), jnp.float32),
            pltpu.SemaphoreType.DMA,
        ],
    )
    def dec(a1_hbm, w_hbm, db_hbm, out_hbm,
            row_v, vals_v, idx_v, vals2_v, idx2_v, rows_v, orow_v,
            bias_v, sem):
        wid = lax.axis_index("s") * 2 + lax.axis_index("c")
        base = wid * tpw
        pltpu.sync_copy(db_hbm, bias_v)
        zf = jnp.zeros((16,), jnp.float32)
        zi = jnp.zeros((16,), jnp.int32)

        def tok_body(t, carry):
            tok = base + t
            pltpu.sync_copy(a1_hbm.at[tok], row_v)
            for i in range(_CAP // 16):
                vals_v[pl.ds(16 * i, 16)] = zf
                idx_v[pl.ds(16 * i, 16)] = zi

            def sb(g, cnt):
                base_e = g * 128
                vs = [row_v[pl.ds(base_e + 16 * i, 16)] for i in range(8)]
                ms = [v != 0.0 for v in vs]
                nz = functools.reduce(jnp.logical_or, ms)
                n_nz = jnp.sum(nz.astype(jnp.int32))

                def do(c):
                    for i in range(8):
                        mi = ms[i].astype(jnp.int32)
                        p = jnp.minimum(plsc.cumsum(mi) + (c - 1),
                                        jnp.int32(_CAP - 1))
                        ii = lax.iota(jnp.int32, 16) + (base_e + 16 * i)
                        plsc.store_scatter(vals_v, [p], vs[i], ms[i])
                        plsc.store_scatter(idx_v, [p], ii, ms[i])
                        c = c + jnp.sum(mi)
                    return c

                return lax.cond(n_nz > 0, do, lambda c: c, cnt)

            cnt = lax.fori_loop(0, BOTTLENECK // 128, sb, jnp.int32(0),
                                unroll=False)

            def trim_cond(c):
                return c > K

            def trim(c):
                chunks = [vals_v[pl.ds(16 * i, 16)]
                          for i in range(_CAP // 16)]
                masked = [jnp.where(ch != 0.0, ch, jnp.inf) for ch in chunks]
                mm = functools.reduce(jnp.minimum, masked)
                m = jnp.min(mm)
                removed = jnp.int32(0)
                for i in range(_CAP // 16):
                    eq = chunks[i] == m
                    vals_v[pl.ds(16 * i, 16)] = jnp.where(
                        eq, 0.0, chunks[i])
                    removed = removed + jnp.sum(eq.astype(jnp.int32))
                return c - removed

            lax.while_loop(trim_cond, trim, cnt)

            vals2_v[pl.ds(0, 16)] = zf
            vals2_v[pl.ds(16, 16)] = zf
            idx2_v[pl.ds(0, 16)] = zi
            idx2_v[pl.ds(16, 16)] = zi
            cc = jnp.int32(0)
            for i in range(_CAP // 16):
                vch = vals_v[pl.ds(16 * i, 16)]
                ich = idx_v[pl.ds(16 * i, 16)]
                m = vch != 0.0
                mi = m.astype(jnp.int32)
                p = jnp.minimum(plsc.cumsum(mi) + (cc - 1), jnp.int32(K - 1))
                plsc.store_scatter(vals2_v, [p], vch, m)
                plsc.store_scatter(idx2_v, [p], ich, m)
                cc = cc + jnp.sum(mi)

            pltpu.async_copy(w_hbm.at[idx2_v], rows_v, sem).wait()

            def j_body(j, c2):
                off = j * 16
                acc = bias_v[pl.ds(off, 16)]
                for kc in range(K // 16):
                    vch = vals2_v[pl.ds(16 * kc, 16)]
                    for k in range(16):
                        kk = 16 * kc + k
                        acc = acc + (_vbroadcast_lane(vch, k) *
                                     rows_v[kk, pl.ds(off, 16)])
                orow_v[pl.ds(off, 16)] = acc
                return c2

            lax.fori_loop(0, INPUT_DIM // 16, j_body, jnp.int32(0),
                          unroll=False)
            pltpu.sync_copy(orow_v, out_hbm.at[tok])
            return carry

        lax.fori_loop(0, tpw, tok_body, jnp.int32(0), unroll=False)

    return dec(a1, W, db1d)


def _split_bf16(a):
    hi = a.astype(jnp.bfloat16)
    lo = (a - hi.astype(jnp.float32)).astype(jnp.bfloat16)
    return hi, lo


def kernel(x, W, b, dec_bias):
    if x.ndim == 1:
        x = x[None, :]
    n_tok = x.shape[0]
    b2d = b.reshape(1, BOTTLENECK)
    db2d = dec_bias.reshape(1, INPUT_DIM)
    tok_blk_mm = min(512, n_tok)
    if n_tok % (4 * _NW * 128) != 0:
        z1 = _encoder_f32(x, W, b2d, tok_blk_mm, 1024)
        a1 = _topk_mask(z1, min(128, n_tok))
        return _decoder(a1.astype(jnp.bfloat16), W.astype(jnp.bfloat16),
                        db2d, tok_blk_mm, 2048)

    n_q = 4
    qs = n_tok // n_q
    outs = []
    for q in range(n_q):
        sl = slice(q * qs, (q + 1) * qs)
        zq = _encoder_f32(x[sl], W, b2d, min(512, qs), 1024)
        aq = _topk_mask(zq, min(128, qs), exact=False)
        outs.append(_sc_decoder(aq, W, dec_bias))
    return jnp.concatenate(outs, axis=0)


# speedup vs baseline: 1.0005x; 1.0005x over previous
import functools

import jax
import jax.numpy as jnp
from jax import lax
from jax.experimental import pallas as pl
from jax.experimental.pallas import tpu as pltpu
from jax.experimental.pallas import tpu_sc as plsc

INPUT_DIM = 2048
BOTTLENECK = 16384
K = 32


def _enc_body_f32(w_ref, x_ref, b_ref, z1_ref):
    z = jax.lax.dot_general(
        x_ref[...], w_ref[...],
        dimension_numbers=(((1,), (1,)), ((), ())),
        preferred_element_type=jnp.float32)
    z1_ref[...] = z + b_ref[...]


def _encoder_f32(x, W, b2d, tok_blk, bn_blk):
    n_tok = x.shape[0]
    grid = (BOTTLENECK // bn_blk, n_tok // tok_blk)
    return pl.pallas_call(
        _enc_body_f32,
        grid=grid,
        in_specs=[
            pl.BlockSpec((bn_blk, INPUT_DIM), lambda j, i: (j, 0)),
            pl.BlockSpec((tok_blk, INPUT_DIM), lambda j, i: (i, 0)),
            pl.BlockSpec((1, bn_blk), lambda j, i: (0, j)),
        ],
        out_specs=pl.BlockSpec((tok_blk, bn_blk), lambda j, i: (i, j)),
        out_shape=jax.ShapeDtypeStruct((n_tok, BOTTLENECK), jnp.float32),
        compiler_params=pltpu.CompilerParams(
            dimension_semantics=("arbitrary", "arbitrary"),
        ),
    )(W, x, b2d)


def _enc_body(wh_ref, wl_ref, xh_ref, xl_ref, b_ref, z1_ref):
    dn = (((1,), (1,)), ((), ()))
    z = jax.lax.dot_general(
        xh_ref[...], wh_ref[...], dimension_numbers=dn,
        preferred_element_type=jnp.float32)
    z += jax.lax.dot_general(
        xh_ref[...], wl_ref[...], dimension_numbers=dn,
        preferred_element_type=jnp.float32)
    z += jax.lax.dot_general(
        xl_ref[...], wh_ref[...], dimension_numbers=dn,
        preferred_element_type=jnp.float32)
    z1_ref[...] = z + b_ref[...]


def _encoder(xh, xl, Wh, Wl, b2d, tok_blk, bn_blk):
    n_tok = xh.shape[0]
    grid = (BOTTLENECK // bn_blk, n_tok // tok_blk)
    w_spec = pl.BlockSpec((bn_blk, INPUT_DIM), lambda j, i: (j, 0))
    x_spec = pl.BlockSpec((tok_blk, INPUT_DIM), lambda j, i: (i, 0))
    return pl.pallas_call(
        _enc_body,
        grid=grid,
        in_specs=[w_spec, w_spec, x_spec, x_spec,
                  pl.BlockSpec((1, bn_blk), lambda j, i: (0, j))],
        out_specs=pl.BlockSpec((tok_blk, bn_blk), lambda j, i: (i, j)),
        out_shape=jax.ShapeDtypeStruct((n_tok, BOTTLENECK), jnp.float32),
        compiler_params=pltpu.CompilerParams(
            dimension_semantics=("arbitrary", "arbitrary"),
        ),
    )(Wh, Wl, xh, xl, b2d)


def _topk_body_ablate(z1_ref, a1_ref, fs_ref):
    a1_ref[...] = z1_ref[...].astype(jnp.bfloat16)


def _topk_body(z1_ref, a1_ref, fs_ref, *, out_dtype=jnp.float32,
               exact=True, thr_only=False):
    R = z1_ref.shape[0]
    N = z1_ref.shape[1]
    NEG = jnp.float32(-jnp.inf)

    fs_ref[:, : N // 2] = jnp.maximum(z1_ref[:, : N // 2], z1_ref[:, N // 2:])
    w = N // 4
    while w >= 128:
        fs_ref[:, :w] = jnp.maximum(fs_ref[:, :w], fs_ref[:, w:2 * w])
        w //= 2

    def extract(n_iter, width):
        def body(_, t):
            blk = fs_ref[:, :width]
            m = jnp.max(blk, axis=1, keepdims=True)
            fs_ref[:, :width] = jnp.where(blk >= m, NEG, blk)
            return m
        return jax.lax.fori_loop(
            0, n_iter, body, jnp.zeros((R, 1), jnp.float32))

    t_lo = extract(K + 1, 128)

    zlo = z1_ref[:, : N // 2]
    zhi = z1_ref[:, N // 2:]
    fs_ref[:, : N // 2] = jnp.maximum(
        jnp.where(zlo >= t_lo, zlo, NEG), jnp.where(zhi >= t_lo, zhi, NEG))
    w = N // 4
    while w >= 1024:
        fs_ref[:, :w] = jnp.maximum(fs_ref[:, :w], fs_ref[:, w:2 * w])
        w //= 2

    thr = jnp.maximum(extract(K, 1024), t_lo)
    if exact:
        cnt1 = jnp.sum((z1_ref[...] >= thr).astype(jnp.float32),
                       axis=1, keepdims=True)
        kf = jnp.float32(K)

        def cond(carry):
            _, cnt = carry
            return jnp.any(cnt > kf)

        def body(carry):
            t, cnt = carry
            zz = z1_ref[...]
            active = cnt > kf
            m = jnp.min(jnp.where(zz >= t, zz, jnp.inf),
                        axis=1, keepdims=True)
            mb = jax.lax.bitcast_convert_type(m, jnp.uint32)
            up = jnp.where(m >= 0, mb + 1, mb - 1)
            t_next = jax.lax.bitcast_convert_type(up, jnp.float32)
            t_new = jnp.where(active, t_next, t)
            cnt_new = jnp.sum((zz >= t_new).astype(jnp.float32),
                              axis=1, keepdims=True)
            return t_new, cnt_new

        thr, _ = jax.lax.while_loop(cond, body, (thr, cnt1))
    if thr_only:
        a1_ref[...] = thr
    else:
        z = z1_ref[...]
        a1_ref[...] = jnp.where(z >= thr, z, 0.0).astype(out_dtype)


def _topk_thr(z1, tok_blk, exact=False):
    n_tok = z1.shape[0]
    return pl.pallas_call(
        functools.partial(_topk_body, exact=exact, thr_only=True),
        grid=(n_tok // tok_blk,),
        in_specs=[pl.BlockSpec((tok_blk, BOTTLENECK), lambda i: (i, 0))],
        out_specs=pl.BlockSpec((tok_blk, 1), lambda i: (i, 0)),
        out_shape=jax.ShapeDtypeStruct((n_tok, 1), jnp.float32),
        scratch_shapes=[pltpu.VMEM((tok_blk, BOTTLENECK // 2), jnp.float32)],
        compiler_params=pltpu.CompilerParams(
            dimension_semantics=("arbitrary",),
        ),
    )(z1)


def _topk_mask(z1, tok_blk, out_dtype=jnp.float32, exact=True):
    n_tok = z1.shape[0]
    return pl.pallas_call(
        functools.partial(_topk_body, out_dtype=out_dtype, exact=exact),
        grid=(n_tok // tok_blk,),
        in_specs=[pl.BlockSpec((tok_blk, BOTTLENECK), lambda i: (i, 0))],
        out_specs=pl.BlockSpec((tok_blk, BOTTLENECK), lambda i: (i, 0)),
        out_shape=jax.ShapeDtypeStruct((n_tok, BOTTLENECK), out_dtype),
        scratch_shapes=[pltpu.VMEM((tok_blk, BOTTLENECK // 2), jnp.float32)],
        compiler_params=pltpu.CompilerParams(
            dimension_semantics=("arbitrary",),
        ),
    )(z1)


def _dec_body(a1_ref, w_ref, db_ref, z2_ref, acc_ref, *, n_kc):
    kc = pl.program_id(1)

    @pl.when(kc == 0)
    def _():
        acc_ref[...] = jnp.zeros_like(acc_ref)

    acc_ref[...] += jax.lax.dot_general(
        a1_ref[...], w_ref[...],
        dimension_numbers=(((1,), (0,)), ((), ())),
        preferred_element_type=jnp.float32,
    )

    @pl.when(kc == n_kc - 1)
    def _():
        z2_ref[...] = acc_ref[...] + db_ref[...]


def _decoder(a1, W, db2d, tok_blk, kc_blk):
    n_tok = a1.shape[0]
    n_kc = BOTTLENECK // kc_blk
    grid = (n_tok // tok_blk, n_kc)
    return pl.pallas_call(
        functools.partial(_dec_body, n_kc=n_kc),
        grid=grid,
        in_specs=[
            pl.BlockSpec((tok_blk, kc_blk), lambda i, k: (i, k)),
            pl.BlockSpec((kc_blk, INPUT_DIM), lambda i, k: (k, 0)),
            pl.BlockSpec((1, INPUT_DIM), lambda i, k: (0, 0)),
        ],
        out_specs=pl.BlockSpec((tok_blk, INPUT_DIM), lambda i, k: (i, 0)),
        out_shape=jax.ShapeDtypeStruct((n_tok, INPUT_DIM), jnp.float32),
        scratch_shapes=[pltpu.VMEM((tok_blk, INPUT_DIM), jnp.float32)],
        compiler_params=pltpu.CompilerParams(
            dimension_semantics=("arbitrary", "arbitrary"),
        ),
    )(a1, W, db2d)


_NW = 32


def _vbroadcast_lane(v, lane):
    idx = jnp.full((16,), lane, dtype=jnp.int32)
    return lax.gather(
        v, idx[:, None],
        dimension_numbers=lax.GatherDimensionNumbers(
            offset_dims=(), collapsed_slice_dims=(0,), start_index_map=(0,)),
        slice_sizes=(1,),
        mode=lax.GatherScatterMode.PROMISE_IN_BOUNDS)


_CAP = 48


def _sc_decoder(z1, t1, W, db1d):
    n_tok = z1.shape[0]
    tpw = n_tok // _NW
    mesh = plsc.VectorSubcoreMesh(core_axis_name="c", subcore_axis_name="s")

    @functools.partial(
        pl.kernel,
        mesh=mesh,
        out_type=jax.ShapeDtypeStruct((n_tok, INPUT_DIM), jnp.float32),
        scratch_types=[
            pltpu.VMEM((BOTTLENECK,), jnp.float32),
            pltpu.VMEM((_CAP,), jnp.float32),
            pltpu.VMEM((_CAP,), jnp.int32),
            pltpu.VMEM((K,), jnp.float32),
            pltpu.VMEM((K,), jnp.int32),
            pltpu.VMEM((K, INPUT_DIM), jnp.float32),
            pltpu.VMEM((INPUT_DIM,), jnp.float32),
            pltpu.VMEM((INPUT_DIM,), jnp.float32),
            pltpu.VMEM((max(16, n_tok // _NW),), jnp.float32),
            pltpu.SemaphoreType.DMA,
        ],
    )
    def dec(z1_hbm, t1_hbm, w_hbm, db_hbm, out_hbm,
            row_v, vals_v, idx_v, vals2_v, idx2_v, rows_v, orow_v,
            bias_v, th_v, sem):
        wid = lax.axis_index("s") * 2 + lax.axis_index("c")
        base = wid * tpw
        pltpu.sync_copy(db_hbm, bias_v)
        pltpu.sync_copy(t1_hbm.at[pl.ds(base, tpw)], th_v)
        zf = jnp.zeros((16,), jnp.float32)
        zi = jnp.zeros((16,), jnp.int32)

        def tok_body(t, carry):
            tok = base + t
            pltpu.sync_copy(z1_hbm.at[tok], row_v)
            tch = th_v[pl.ds((t // 16) * 16, 16)]
            tsplat = _vbroadcast_lane(tch, t % 16)
            for i in range(_CAP // 16):
                vals_v[pl.ds(16 * i, 16)] = zf
                idx_v[pl.ds(16 * i, 16)] = zi

            def sb(g, cnt):
                base_e = g * 128
                vs = [row_v[pl.ds(base_e + 16 * i, 16)] for i in range(8)]
                ms = [v >= tsplat for v in vs]
                nz = functools.reduce(jnp.logical_or, ms)
                n_nz = jnp.sum(nz.astype(jnp.int32))

                def do(c):
                    for i in range(8):
                        mi = ms[i].astype(jnp.int32)
                        p = jnp.minimum(plsc.cumsum(mi) + (c - 1),
                                        jnp.int32(_CAP - 1))
                        ii = lax.iota(jnp.int32, 16) + (base_e + 16 * i)
                        plsc.store_scatter(vals_v, [p], vs[i], ms[i])
                        plsc.store_scatter(idx_v, [p], ii, ms[i])
                        c = c + jnp.sum(mi)
                    return c

                return lax.cond(n_nz > 0, do, lambda c: c, cnt)

            cnt = lax.fori_loop(0, BOTTLENECK // 128, sb, jnp.int32(0),
                                unroll=False)

            def trim_cond(c):
                return c > K

            def trim(c):
                chunks = [vals_v[pl.ds(16 * i, 16)]
                          for i in range(_CAP // 16)]
                masked = [jnp.where(ch != 0.0, ch, jnp.inf) for ch in chunks]
                mm = functools.reduce(jnp.minimum, masked)
                m = jnp.min(mm)
                removed = jnp.int32(0)
                for i in range(_CAP // 16):
                    eq = chunks[i] == m
                    vals_v[pl.ds(16 * i, 16)] = jnp.where(
                        eq, 0.0, chunks[i])
                    removed = removed + jnp.sum(eq.astype(jnp.int32))
                return c - removed

            lax.while_loop(trim_cond, trim, cnt)

            vals2_v[pl.ds(0, 16)] = zf
            vals2_v[pl.ds(16, 16)] = zf
            idx2_v[pl.ds(0, 16)] = zi
            idx2_v[pl.ds(16, 16)] = zi
            cc = jnp.int32(0)
            for i in range(_CAP // 16):
                vch = vals_v[pl.ds(16 * i, 16)]
                ich = idx_v[pl.ds(16 * i, 16)]
                m = vch != 0.0
                mi = m.astype(jnp.int32)
                p = jnp.minimum(plsc.cumsum(mi) + (cc - 1), jnp.int32(K - 1))
                plsc.store_scatter(vals2_v, [p], vch, m)
                plsc.store_scatter(idx2_v, [p], ich, m)
                cc = cc + jnp.sum(mi)

            pltpu.async_copy(w_hbm.at[idx2_v], rows_v, sem).wait()

            def j_body(j, c2):
                off = j * 16
                acc = bias_v[pl.ds(off, 16)]
                for kc in range(K // 16):
                    vch = vals2_v[pl.ds(16 * kc, 16)]
                    for k in range(16):
                        kk = 16 * kc + k
                        acc = acc + (_vbroadcast_lane(vch, k) *
                                     rows_v[kk, pl.ds(off, 16)])
                orow_v[pl.ds(off, 16)] = acc
                return c2

            lax.fori_loop(0, INPUT_DIM // 16, j_body, jnp.int32(0),
                          unroll=False)
            pltpu.sync_copy(orow_v, out_hbm.at[tok])
            return carry

        lax.fori_loop(0, tpw, tok_body, jnp.int32(0), unroll=False)

    return dec(z1, t1, W, db1d)


def _split_bf16(a):
    hi = a.astype(jnp.bfloat16)
    lo = (a - hi.astype(jnp.float32)).astype(jnp.bfloat16)
    return hi, lo


def kernel(x, W, b, dec_bias):
    if x.ndim == 1:
        x = x[None, :]
    n_tok = x.shape[0]
    b2d = b.reshape(1, BOTTLENECK)
    db2d = dec_bias.reshape(1, INPUT_DIM)
    tok_blk_mm = min(512, n_tok)
    if n_tok % (4 * _NW * 128) != 0:
        z1 = _encoder_f32(x, W, b2d, tok_blk_mm, 1024)
        a1 = _topk_mask(z1, min(128, n_tok))
        return _decoder(a1.astype(jnp.bfloat16), W.astype(jnp.bfloat16),
                        db2d, tok_blk_mm, 2048)

    n_q = 4
    qs = n_tok // n_q
    outs = []
    for q in range(n_q):
        sl = slice(q * qs, (q + 1) * qs)
        zq = _encoder_f32(x[sl], W, b2d, min(512, qs), 1024)
        tq = _topk_thr(zq, min(128, qs)).reshape(qs)
        outs.append(_sc_decoder(zq, tq, W, dec_bias))
    return jnp.concatenate(outs, axis=0)
